# Initial kernel scaffold; baseline (speedup 1.0000x reference)
import jax
import jax.numpy as jnp
from jax.experimental import pallas as pl


def kernel(inputs, sc_ind):
    raise NotImplementedError("write your pallas kernel here")


# speedup vs baseline: 1.4463x; 1.4463x over previous
import jax
import jax.numpy as jnp
from jax import lax
from jax.experimental import pallas as pl
from jax.experimental.pallas import tpu as pltpu
from jax.experimental.pallas import tpu_sc as plsc

ROWS = 16 * 4 * 2 * 14
COLS = 4096
NSC = 3276
LANES = 16
NGRP = (NSC + LANES - 1) // LANES
NSC_PAD = NGRP * LANES
OUT_PAD = 3328
NUM_CORES = 2
NUM_SUBCORES = 16
NW = NUM_CORES * NUM_SUBCORES
R = 8
NCHUNK = ROWS // R


def _body(x_hbm, idx_hbm, out_hbm, idx_v, row_v, stage_v):
    wid = lax.axis_index("s") * NUM_CORES + lax.axis_index("c")

    pltpu.sync_copy(idx_hbm, idx_v)

    for k in range(NCHUNK // NW):
        row0 = (wid + NW * k) * R
        pltpu.sync_copy(x_hbm.at[pl.ds(row0 * COLS, R * COLS)], row_v)
        for r in range(R):
            roff = jnp.int32(r * COLS)

            def grp(j, c, roff=roff, r=r):
                o = j * LANES
                iv = idx_v[pl.ds(o, LANES)]
                stage_v[r, pl.ds(o, LANES)] = plsc.load_gather(
                    row_v, [iv + roff]
                )
                return c

            lax.fori_loop(0, NGRP, grp, jnp.int32(0))
        pltpu.sync_copy(stage_v, out_hbm.at[pl.ds(row0, R)])


_gather = pl.kernel(
    _body,
    out_type=jax.ShapeDtypeStruct((ROWS, OUT_PAD), jnp.float32),
    mesh=plsc.VectorSubcoreMesh(core_axis_name="c", subcore_axis_name="s"),
    scratch_types=[
        pltpu.VMEM((NSC_PAD,), jnp.int32),
        pltpu.VMEM((R * COLS,), jnp.float32),
        pltpu.VMEM((R, OUT_PAD), jnp.float32),
    ],
    compiler_params=pltpu.CompilerParams(needs_layout_passes=False),
)


@jax.jit
def kernel(inputs, sc_ind):
    x = inputs.reshape(-1)
    idx = jnp.concatenate(
        [sc_ind.astype(jnp.int32), jnp.zeros((NSC_PAD - NSC,), jnp.int32)]
    )
    out = _gather(x, idx)
    return out[:, :NSC].reshape(16, 4, 2, 14, NSC)


# speedup vs baseline: 2.3543x; 1.6279x over previous
import jax
import jax.numpy as jnp
from jax import lax
from jax.experimental import pallas as pl
from jax.experimental.pallas import tpu as pltpu
from jax.experimental.pallas import tpu_sc as plsc

ROWS = 16 * 4 * 2 * 14
COLS = 4096
NSC = 3276
LANES = 16
NGRP = (NSC + LANES - 1) // LANES
NSC_PAD = NGRP * LANES
OUT_PAD = 3328
NUM_CORES = 2
NUM_SUBCORES = 16
NW = NUM_CORES * NUM_SUBCORES
R = 8
NCHUNK = ROWS // R


def _body(x_hbm, idx_hbm, out_hbm, idx_v, row_v, stage_v):
    wid = lax.axis_index("s") * NUM_CORES + lax.axis_index("c")

    pltpu.sync_copy(idx_hbm, idx_v)

    for k in range(NCHUNK // NW):
        row0 = (wid + NW * k) * R
        pltpu.sync_copy(x_hbm.at[pl.ds(row0 * COLS, R * COLS)], row_v)

        def grp(j):
            o = j * LANES
            iv = idx_v[pl.ds(o, LANES)]
            for r in range(R):
                stage_v[r, pl.ds(o, LANES)] = plsc.load_gather(
                    row_v, [iv + jnp.int32(r * COLS)]
                )

        plsc.parallel_loop(0, NGRP, 1, unroll=4)(grp)
        pltpu.sync_copy(stage_v, out_hbm.at[pl.ds(row0, R)])


_gather = pl.kernel(
    _body,
    out_type=jax.ShapeDtypeStruct((ROWS, OUT_PAD), jnp.float32),
    mesh=plsc.VectorSubcoreMesh(core_axis_name="c", subcore_axis_name="s"),
    scratch_types=[
        pltpu.VMEM((NSC_PAD,), jnp.int32),
        pltpu.VMEM((R * COLS,), jnp.float32),
        pltpu.VMEM((R, OUT_PAD), jnp.float32),
    ],
    compiler_params=pltpu.CompilerParams(needs_layout_passes=False),
)


@jax.jit
def kernel(inputs, sc_ind):
    x = inputs.reshape(-1)
    idx = jnp.concatenate(
        [sc_ind.astype(jnp.int32), jnp.zeros((NSC_PAD - NSC,), jnp.int32)]
    )
    out = _gather(x, idx)
    return out[:, :NSC].reshape(16, 4, 2, 14, NSC)
